# baseline (speedup 1.0000x reference)
import functools

import jax
import jax.numpy as jnp
from jax import lax
from jax.experimental import pallas as pl
from jax.experimental.pallas import tpu as pltpu
from jax.experimental.pallas import tpu_sc as plsc

N1 = 16384
N2 = 4096
C = 512
K = 5
KP = 8


BN_T1 = 256


def _topk_body(p1_ref, p2t_ref, idx_ref):
    p1 = p1_ref[...]
    p2t = p2t_ref[...]
    mm = jnp.dot(p1.astype(jnp.bfloat16), p2t.astype(jnp.bfloat16),
                 preferred_element_type=jnp.float32)
    s1 = ((p1[:, 0:1] * p1[:, 0:1] + p1[:, 1:2] * p1[:, 1:2])
          + p1[:, 2:3] * p1[:, 2:3])
    s2 = ((p2t[0:1, :] * p2t[0:1, :] + p2t[1:2, :] * p2t[1:2, :])
          + p2t[2:3, :] * p2t[2:3, :])
    d = (s1 - 2.0 * mm) + s2
    iota = lax.broadcasted_iota(jnp.int32, (BN_T1, N2), 1)
    idxs = []
    for k in range(K):
        am = jnp.argmin(d, axis=1).astype(jnp.int32)[:, None]
        idxs.append(am)
        if k < K - 1:
            d = jnp.where(iota == am, jnp.float32(jnp.inf), d)
    ik = jnp.concatenate(idxs, axis=1)
    idx_ref[...] = jnp.concatenate(
        [ik, jnp.zeros((BN_T1, KP - K), jnp.int32)], axis=1)


def _topk_call(p1, p2t):
    nq = p1.shape[0]
    return pl.pallas_call(
        _topk_body,
        grid=(nq // BN_T1,),
        in_specs=[
            pl.BlockSpec((BN_T1, 3), lambda i: (i, 0)),
            pl.BlockSpec((3, N2), lambda i: (0, 0)),
        ],
        out_specs=pl.BlockSpec((BN_T1, KP), lambda i: (i, 0)),
        out_shape=jax.ShapeDtypeStruct((nq, KP), jnp.int32),
    )(p1, p2t)


BM_T2 = 512


def _lin1_body(x1_ref, w1t_ref, b1_ref, g1_ref, be1_ref, y_ref, st_ref):
    i = pl.program_id(0)
    y = jnp.dot(x1_ref[...].astype(jnp.bfloat16),
                w1t_ref[...].astype(jnp.bfloat16),
                preferred_element_type=jnp.float32) + b1_ref[...]
    y_ref[...] = y

    @pl.when(i == 0)
    def _():
        st_ref[...] = jnp.zeros_like(st_ref)

    st_ref[0:1, :] += jnp.sum(y, axis=0, keepdims=True)
    st_ref[1:2, :] += jnp.sum(y * y, axis=0, keepdims=True)

    @pl.when(i == pl.num_programs(0) - 1)
    def _():
        mean = st_ref[0:1, :] / jnp.float32(N1)
        var = st_ref[1:2, :] / jnp.float32(N1) - mean * mean
        st_ref[0:1, :] = mean
        st_ref[1:2, :] = g1_ref[...] * lax.rsqrt(var + jnp.float32(1e-5))
        st_ref[2:3, :] = be1_ref[...]


def _lin1_call(x1, w1t, b1, g1, be1):
    return pl.pallas_call(
        _lin1_body,
        grid=(N1 // BM_T2,),
        in_specs=[
            pl.BlockSpec((BM_T2, 2 * C), lambda i: (i, 0)),
            pl.BlockSpec((2 * C, C), lambda i: (0, 0)),
            pl.BlockSpec((1, C), lambda i: (0, 0)),
            pl.BlockSpec((1, C), lambda i: (0, 0)),
            pl.BlockSpec((1, C), lambda i: (0, 0)),
        ],
        out_specs=[
            pl.BlockSpec((BM_T2, C), lambda i: (i, 0)),
            pl.BlockSpec((8, C), lambda i: (0, 0)),
        ],
        out_shape=[
            jax.ShapeDtypeStruct((N1, C), jnp.float32),
            jax.ShapeDtypeStruct((8, C), jnp.float32),
        ],
    )(x1, w1t, b1, g1, be1)


BM_T3 = 512


def _lin2_body(x2_ref, w2t_ref, b2_ref, z_ref):
    z_ref[...] = jnp.dot(x2_ref[...], w2t_ref[...],
                         preferred_element_type=jnp.float32) + b2_ref[...]


def _lin2_call(x2, w2t, b2):
    return pl.pallas_call(
        _lin2_body,
        grid=(N2 // BM_T3,),
        in_specs=[
            pl.BlockSpec((BM_T3, C), lambda i: (i, 0)),
            pl.BlockSpec((C, C), lambda i: (0, 0)),
            pl.BlockSpec((1, C), lambda i: (0, 0)),
        ],
        out_specs=pl.BlockSpec((BM_T3, C), lambda i: (i, 0)),
        out_shape=jax.ShapeDtypeStruct((N2, C), jnp.float32),
    )(x2, w2t, b2)


NW = 32
QB = 16


@functools.cache
def _interp_call(nq):
    bpw = nq // NW
    nsb = bpw // QB

    def _interp_body(z2_hbm, idxf_hbm, p1f_hbm, p2tf_hbm, out_hbm,
                     idx_v, p1_v, p2t_v, wtmp_v, rows_v, acc_v, sem):
        cc = lax.axis_index("c")
        ss = lax.axis_index("s")
        wid = ss * 2 + cc
        base = wid * bpw
        pltpu.sync_copy(idxf_hbm.at[pl.ds(base * KP, bpw * KP)], idx_v)
        pltpu.sync_copy(p1f_hbm.at[pl.ds(base * 3, bpw * 3)], p1_v)
        pltpu.sync_copy(p2tf_hbm, p2t_v)
        iota16 = lax.broadcasted_iota(jnp.int32, (16,), 0)

        def fire_gathers(b, buf):
            qb = b * QB
            for k in range(K):
                pos = (qb + iota16) * KP + k
                cidx = plsc.load_gather(idx_v, [pos])
                pltpu.async_copy(z2_hbm.at[cidx], rows_v.at[buf, k], sem)

        fire_gathers(0, 0)

        def sb_body(b, carry):
            qb = b * QB
            cur = lax.rem(b, 2)
            for k in range(K):
                pltpu.make_async_copy(
                    z2_hbm.at[pl.ds(0, QB)], rows_v.at[cur, k], sem).wait()
            @pl.when(b + 1 < nsb)
            def _():
                fire_gathers(b + 1, 1 - cur)
            p1x = plsc.load_gather(p1_v, [(qb + iota16) * 3 + 0])
            p1y = plsc.load_gather(p1_v, [(qb + iota16) * 3 + 1])
            p1z = plsc.load_gather(p1_v, [(qb + iota16) * 3 + 2])
            wks = []
            for k in range(K):
                pos = (qb + iota16) * KP + k
                cidx = plsc.load_gather(idx_v, [pos])
                dx = plsc.load_gather(p2t_v, [cidx]) - p1x
                dy = plsc.load_gather(p2t_v, [cidx + N2]) - p1y
                dz = plsc.load_gather(p2t_v, [cidx + 2 * N2]) - p1z
                d16 = (dx * dx + dy * dy) + dz * dz
                d16 = jnp.maximum(d16, jnp.float32(1e-10))
                wks.append(1.0 / d16)
            wsum = ((wks[0] + wks[1]) + (wks[2] + wks[3])) + wks[4]
            for k in range(K):
                wtmp_v[pl.ds(k * 16, 16)] = wks[k] / wsum

            def q_body(q, carry2):
                ws = []
                for k in range(K):
                    wk = plsc.load_gather(
                        wtmp_v, [jnp.broadcast_to(k * 16 + q, (16,))])
                    ws.append(wk)
                for j in range(C // 16):
                    acc = ws[0] * rows_v[cur, 0, q, pl.ds(j * 16, 16)]
                    for k in range(1, K):
                        acc = acc + ws[k] * rows_v[cur, k, q, pl.ds(j * 16, 16)]
                    acc_v[pl.ds(q * C + j * 16, 16)] = acc
                return carry2

            lax.fori_loop(0, QB, q_body, 0)
            pltpu.sync_copy(acc_v, out_hbm.at[pl.ds((base + qb) * C, QB * C)])
            return carry

        lax.fori_loop(0, nsb, sb_body, 0)

    return pl.kernel(
        _interp_body,
        out_type=jax.ShapeDtypeStruct((nq * C,), jnp.float32),
        mesh=plsc.VectorSubcoreMesh(core_axis_name="c", subcore_axis_name="s"),
        compiler_params=pltpu.CompilerParams(needs_layout_passes=False),
        scratch_types=[
            pltpu.VMEM((bpw * KP,), jnp.int32),
            pltpu.VMEM((bpw * 3,), jnp.float32),
            pltpu.VMEM((3 * N2,), jnp.float32),
            pltpu.VMEM((K * QB,), jnp.float32),
            pltpu.VMEM((2, K, QB, C), jnp.float32),
            pltpu.VMEM((QB * C,), jnp.float32),
            pltpu.SemaphoreType.DMA,
        ],
    )


BM_T5 = 512


def _final_body(y_ref, it_ref, st_ref, o_ref):
    mean = st_ref[0:1, :]
    scale = st_ref[1:2, :]
    be = st_ref[2:3, :]
    h1 = jnp.maximum((y_ref[...] - mean) * scale + be, 0.0)
    h2 = jnp.maximum(it_ref[...], 0.0)
    o_ref[...] = h1 + h2


def _final_call(y1, interp, stats):
    return pl.pallas_call(
        _final_body,
        grid=(N1 // BM_T5,),
        in_specs=[
            pl.BlockSpec((BM_T5, C), lambda i: (i, 0)),
            pl.BlockSpec((BM_T5, C), lambda i: (i, 0)),
            pl.BlockSpec((8, C), lambda i: (0, 0)),
        ],
        out_specs=pl.BlockSpec((BM_T5, C), lambda i: (i, 0)),
        out_shape=jax.ShapeDtypeStruct((N1, C), jnp.float32),
    )(y1, interp, stats)


def kernel(p1, x1, o1, p2, x2, o2, W1, b1, g1, be1, W2, b2):
    p2t = p2.T
    p2tf = p2t.reshape(-1)
    z2 = _lin2_call(x2, W2.T, b2[None, :])
    H = N1 // 4
    halves = []
    for lo in range(0, N1, H):
        p1h = p1[lo:lo + H]
        idx8 = _topk_call(p1h, p2t)
        halves.append(_interp_call(H)(
            z2, idx8.reshape(-1), p1h.reshape(-1), p2tf))
    y1, stats = _lin1_call(x1, W1.T, b1[None, :], g1[None, :], be1[None, :])
    interp = jnp.concatenate(
        [h.reshape(H, C) for h in halves], axis=0)
    return _final_call(y1, interp, stats)


# speedup vs baseline: 1.0026x; 1.0026x over previous
import functools

import jax
import jax.numpy as jnp
from jax import lax
from jax.experimental import pallas as pl
from jax.experimental.pallas import tpu as pltpu
from jax.experimental.pallas import tpu_sc as plsc

N1 = 16384
N2 = 4096
C = 512
K = 5
KP = 8


BN_T1 = 256


def _topk_body(p1_ref, p2t_ref, idx_ref):
    p1 = p1_ref[...]
    p2t = p2t_ref[...]
    mm = jnp.dot(p1.astype(jnp.bfloat16), p2t.astype(jnp.bfloat16),
                 preferred_element_type=jnp.float32)
    s1 = ((p1[:, 0:1] * p1[:, 0:1] + p1[:, 1:2] * p1[:, 1:2])
          + p1[:, 2:3] * p1[:, 2:3])
    s2 = ((p2t[0:1, :] * p2t[0:1, :] + p2t[1:2, :] * p2t[1:2, :])
          + p2t[2:3, :] * p2t[2:3, :])
    d = (s1 - 2.0 * mm) + s2
    iota = lax.broadcasted_iota(jnp.int32, (BN_T1, N2), 1)
    idxs = []
    for k in range(K):
        am = jnp.argmin(d, axis=1).astype(jnp.int32)[:, None]
        idxs.append(am)
        if k < K - 1:
            d = jnp.where(iota == am, jnp.float32(jnp.inf), d)
    ik = jnp.concatenate(idxs, axis=1)
    idx_ref[...] = jnp.concatenate(
        [ik, jnp.zeros((BN_T1, KP - K), jnp.int32)], axis=1)


def _topk_call(p1, p2t):
    nq = p1.shape[0]
    return pl.pallas_call(
        _topk_body,
        grid=(nq // BN_T1,),
        in_specs=[
            pl.BlockSpec((BN_T1, 3), lambda i: (i, 0)),
            pl.BlockSpec((3, N2), lambda i: (0, 0)),
        ],
        out_specs=pl.BlockSpec((BN_T1, KP), lambda i: (i, 0)),
        out_shape=jax.ShapeDtypeStruct((nq, KP), jnp.int32),
    )(p1, p2t)


BM_T2 = 512


def _lin1_body(x1_ref, w1t_ref, b1_ref, g1_ref, be1_ref, y_ref, st_ref):
    i = pl.program_id(0)
    y = jnp.dot(x1_ref[...].astype(jnp.bfloat16),
                w1t_ref[...].astype(jnp.bfloat16),
                preferred_element_type=jnp.float32) + b1_ref[...]
    y_ref[...] = y

    @pl.when(i == 0)
    def _():
        st_ref[...] = jnp.zeros_like(st_ref)

    st_ref[0:1, :] += jnp.sum(y, axis=0, keepdims=True)
    st_ref[1:2, :] += jnp.sum(y * y, axis=0, keepdims=True)

    @pl.when(i == pl.num_programs(0) - 1)
    def _():
        mean = st_ref[0:1, :] / jnp.float32(N1)
        var = st_ref[1:2, :] / jnp.float32(N1) - mean * mean
        st_ref[0:1, :] = mean
        st_ref[1:2, :] = g1_ref[...] * lax.rsqrt(var + jnp.float32(1e-5))
        st_ref[2:3, :] = be1_ref[...]


def _lin1_call(x1, w1t, b1, g1, be1):
    return pl.pallas_call(
        _lin1_body,
        grid=(N1 // BM_T2,),
        in_specs=[
            pl.BlockSpec((BM_T2, 2 * C), lambda i: (i, 0)),
            pl.BlockSpec((2 * C, C), lambda i: (0, 0)),
            pl.BlockSpec((1, C), lambda i: (0, 0)),
            pl.BlockSpec((1, C), lambda i: (0, 0)),
            pl.BlockSpec((1, C), lambda i: (0, 0)),
        ],
        out_specs=[
            pl.BlockSpec((BM_T2, C), lambda i: (i, 0)),
            pl.BlockSpec((8, C), lambda i: (0, 0)),
        ],
        out_shape=[
            jax.ShapeDtypeStruct((N1, C), jnp.float32),
            jax.ShapeDtypeStruct((8, C), jnp.float32),
        ],
    )(x1, w1t, b1, g1, be1)


BM_T3 = 512


def _lin2_body(x2_ref, w2t_ref, b2_ref, z_ref):
    z_ref[...] = jnp.dot(x2_ref[...], w2t_ref[...],
                         preferred_element_type=jnp.float32) + b2_ref[...]


def _lin2_call(x2, w2t, b2):
    return pl.pallas_call(
        _lin2_body,
        grid=(N2 // BM_T3,),
        in_specs=[
            pl.BlockSpec((BM_T3, C), lambda i: (i, 0)),
            pl.BlockSpec((C, C), lambda i: (0, 0)),
            pl.BlockSpec((1, C), lambda i: (0, 0)),
        ],
        out_specs=pl.BlockSpec((BM_T3, C), lambda i: (i, 0)),
        out_shape=jax.ShapeDtypeStruct((N2, C), jnp.float32),
    )(x2, w2t, b2)


NW = 32
QB = 16


@functools.cache
def _interp_call(nq):
    bpw = nq // NW
    nsb = bpw // QB

    def _interp_body(z2_hbm, idxf_hbm, p1f_hbm, p2tf_hbm, out_hbm,
                     idx_v, p1_v, p2t_v, wtmp_v, rows_v, acc_v, sem):
        cc = lax.axis_index("c")
        ss = lax.axis_index("s")
        wid = ss * 2 + cc
        base = wid * bpw
        pltpu.sync_copy(idxf_hbm.at[pl.ds(base * KP, bpw * KP)], idx_v)
        pltpu.sync_copy(p1f_hbm.at[pl.ds(base * 3, bpw * 3)], p1_v)
        pltpu.sync_copy(p2tf_hbm, p2t_v)
        iota16 = lax.broadcasted_iota(jnp.int32, (16,), 0)

        def fire_gathers(b, buf):
            qb = b * QB
            for k in range(K):
                pos = (qb + iota16) * KP + k
                cidx = plsc.load_gather(idx_v, [pos])
                pltpu.async_copy(z2_hbm.at[cidx], rows_v.at[buf, k], sem)

        fire_gathers(0, 0)

        def sb_body(b, carry):
            qb = b * QB
            cur = lax.rem(b, 2)
            for k in range(K):
                pltpu.make_async_copy(
                    z2_hbm.at[pl.ds(0, QB)], rows_v.at[cur, k], sem).wait()
            @pl.when(b + 1 < nsb)
            def _():
                fire_gathers(b + 1, 1 - cur)
            p1x = plsc.load_gather(p1_v, [(qb + iota16) * 3 + 0])
            p1y = plsc.load_gather(p1_v, [(qb + iota16) * 3 + 1])
            p1z = plsc.load_gather(p1_v, [(qb + iota16) * 3 + 2])
            wks = []
            for k in range(K):
                pos = (qb + iota16) * KP + k
                cidx = plsc.load_gather(idx_v, [pos])
                dx = plsc.load_gather(p2t_v, [cidx]) - p1x
                dy = plsc.load_gather(p2t_v, [cidx + N2]) - p1y
                dz = plsc.load_gather(p2t_v, [cidx + 2 * N2]) - p1z
                d16 = (dx * dx + dy * dy) + dz * dz
                d16 = jnp.maximum(d16, jnp.float32(1e-10))
                wks.append(1.0 / d16)
            wsum = ((wks[0] + wks[1]) + (wks[2] + wks[3])) + wks[4]
            for k in range(K):
                wtmp_v[pl.ds(k * 16, 16)] = wks[k] / wsum

            def q_body(q, carry2):
                ws = []
                for k in range(K):
                    wk = plsc.load_gather(
                        wtmp_v, [jnp.broadcast_to(k * 16 + q, (16,))])
                    ws.append(wk)
                for j in range(C // 16):
                    acc = ws[0] * rows_v[cur, 0, q, pl.ds(j * 16, 16)]
                    for k in range(1, K):
                        acc = acc + ws[k] * rows_v[cur, k, q, pl.ds(j * 16, 16)]
                    acc_v[pl.ds(q * C + j * 16, 16)] = acc
                return carry2

            lax.fori_loop(0, QB, q_body, 0)
            pltpu.sync_copy(acc_v, out_hbm.at[pl.ds((base + qb) * C, QB * C)])
            return carry

        lax.fori_loop(0, nsb, sb_body, 0)

    return pl.kernel(
        _interp_body,
        out_type=jax.ShapeDtypeStruct((nq * C,), jnp.float32),
        mesh=plsc.VectorSubcoreMesh(core_axis_name="c", subcore_axis_name="s"),
        compiler_params=pltpu.CompilerParams(needs_layout_passes=False),
        scratch_types=[
            pltpu.VMEM((bpw * KP,), jnp.int32),
            pltpu.VMEM((bpw * 3,), jnp.float32),
            pltpu.VMEM((3 * N2,), jnp.float32),
            pltpu.VMEM((K * QB,), jnp.float32),
            pltpu.VMEM((2, K, QB, C), jnp.float32),
            pltpu.VMEM((QB * C,), jnp.float32),
            pltpu.SemaphoreType.DMA,
        ],
    )


BM_T5 = 512


def _final_body(y_ref, it_ref, st_ref, o_ref):
    mean = st_ref[0:1, :]
    scale = st_ref[1:2, :]
    be = st_ref[2:3, :]
    h1 = jnp.maximum((y_ref[...] - mean) * scale + be, 0.0)
    h2 = jnp.maximum(it_ref[...], 0.0)
    o_ref[...] = h1 + h2


def _final_call(y1, interp, stats):
    return pl.pallas_call(
        _final_body,
        grid=(N1 // BM_T5,),
        in_specs=[
            pl.BlockSpec((BM_T5, C), lambda i: (i, 0)),
            pl.BlockSpec((BM_T5, C), lambda i: (i, 0)),
            pl.BlockSpec((8, C), lambda i: (0, 0)),
        ],
        out_specs=pl.BlockSpec((BM_T5, C), lambda i: (i, 0)),
        out_shape=jax.ShapeDtypeStruct((N1, C), jnp.float32),
    )(y1, interp, stats)


def kernel(p1, x1, o1, p2, x2, o2, W1, b1, g1, be1, W2, b2):
    p2t = p2.T
    p2tf = p2t.reshape(-1)
    z2 = _lin2_call(x2, W2.T, b2[None, :])
    H = N1 // 2
    halves = []
    for lo in range(0, N1, H):
        p1h = p1[lo:lo + H]
        idx8 = _topk_call(p1h, p2t)
        halves.append(_interp_call(H)(
            z2, idx8.reshape(-1), p1h.reshape(-1), p2tf))
    y1, stats = _lin1_call(x1, W1.T, b1[None, :], g1[None, :], be1[None, :])
    interp = jnp.concatenate(
        [h.reshape(H, C) for h in halves], axis=0)
    return _final_call(y1, interp, stats)


# speedup vs baseline: 1.0884x; 1.0856x over previous
import functools

import jax
import jax.numpy as jnp
from jax import lax
from jax.experimental import pallas as pl
from jax.experimental.pallas import tpu as pltpu
from jax.experimental.pallas import tpu_sc as plsc

N1 = 16384
N2 = 4096
C = 512
CP = C // 2
K = 5
KP = 8


BN_T1 = 256


def _topk_body(p1_ref, p2t_ref, idx_ref):
    p1 = p1_ref[...]
    p2t = p2t_ref[...]
    mm = jnp.dot(p1.astype(jnp.bfloat16), p2t.astype(jnp.bfloat16),
                 preferred_element_type=jnp.float32)
    s1 = ((p1[:, 0:1] * p1[:, 0:1] + p1[:, 1:2] * p1[:, 1:2])
          + p1[:, 2:3] * p1[:, 2:3])
    s2 = ((p2t[0:1, :] * p2t[0:1, :] + p2t[1:2, :] * p2t[1:2, :])
          + p2t[2:3, :] * p2t[2:3, :])
    d = (s1 - 2.0 * mm) + s2
    iota = lax.broadcasted_iota(jnp.int32, (BN_T1, N2), 1)
    idxs = []
    for k in range(K):
        am = jnp.argmin(d, axis=1).astype(jnp.int32)[:, None]
        idxs.append(am)
        if k < K - 1:
            d = jnp.where(iota == am, jnp.float32(jnp.inf), d)
    ik = jnp.concatenate(idxs, axis=1)
    idx_ref[...] = jnp.concatenate(
        [ik, jnp.zeros((BN_T1, KP - K), jnp.int32)], axis=1)


def _topk_call(p1, p2t):
    nq = p1.shape[0]
    return pl.pallas_call(
        _topk_body,
        grid=(nq // BN_T1,),
        in_specs=[
            pl.BlockSpec((BN_T1, 3), lambda i: (i, 0)),
            pl.BlockSpec((3, N2), lambda i: (0, 0)),
        ],
        out_specs=pl.BlockSpec((BN_T1, KP), lambda i: (i, 0)),
        out_shape=jax.ShapeDtypeStruct((nq, KP), jnp.int32),
    )(p1, p2t)


BM_T2 = 512


def _lin1_body(x1_ref, w1t_ref, b1_ref, g1_ref, be1_ref, y_ref, st_ref):
    i = pl.program_id(0)
    y = jnp.dot(x1_ref[...].astype(jnp.bfloat16),
                w1t_ref[...].astype(jnp.bfloat16),
                preferred_element_type=jnp.float32) + b1_ref[...]
    y_ref[...] = y

    @pl.when(i == 0)
    def _():
        st_ref[...] = jnp.zeros_like(st_ref)

    st_ref[0:1, :] += jnp.sum(y, axis=0, keepdims=True)
    st_ref[1:2, :] += jnp.sum(y * y, axis=0, keepdims=True)

    @pl.when(i == pl.num_programs(0) - 1)
    def _():
        mean = st_ref[0:1, :] / jnp.float32(N1)
        var = st_ref[1:2, :] / jnp.float32(N1) - mean * mean
        st_ref[0:1, :] = mean
        st_ref[1:2, :] = g1_ref[...] * lax.rsqrt(var + jnp.float32(1e-5))
        st_ref[2:3, :] = be1_ref[...]


def _lin1_call(x1, w1t, b1, g1, be1):
    return pl.pallas_call(
        _lin1_body,
        grid=(N1 // BM_T2,),
        in_specs=[
            pl.BlockSpec((BM_T2, 2 * C), lambda i: (i, 0)),
            pl.BlockSpec((2 * C, C), lambda i: (0, 0)),
            pl.BlockSpec((1, C), lambda i: (0, 0)),
            pl.BlockSpec((1, C), lambda i: (0, 0)),
            pl.BlockSpec((1, C), lambda i: (0, 0)),
        ],
        out_specs=[
            pl.BlockSpec((BM_T2, C), lambda i: (i, 0)),
            pl.BlockSpec((8, C), lambda i: (0, 0)),
        ],
        out_shape=[
            jax.ShapeDtypeStruct((N1, C), jnp.float32),
            jax.ShapeDtypeStruct((8, C), jnp.float32),
        ],
    )(x1, w1t, b1, g1, be1)


BM_T3 = 512


def _lin2_body(x2_ref, w2t_ref, b2_ref, z_ref):
    z = jnp.dot(x2_ref[...], w2t_ref[...],
                preferred_element_type=jnp.float32) + b2_ref[...]
    zb = z.astype(jnp.bfloat16)
    lo = lax.bitcast_convert_type(zb[:, :C // 2], jnp.uint16).astype(jnp.int32)
    hi = lax.bitcast_convert_type(zb[:, C // 2:], jnp.uint16).astype(jnp.int32)
    z_ref[...] = lo | (hi << 16)


def _lin2_call(x2, w2t, b2):
    return pl.pallas_call(
        _lin2_body,
        grid=(N2 // BM_T3,),
        in_specs=[
            pl.BlockSpec((BM_T3, C), lambda i: (i, 0)),
            pl.BlockSpec((C, C), lambda i: (0, 0)),
            pl.BlockSpec((1, C), lambda i: (0, 0)),
        ],
        out_specs=pl.BlockSpec((BM_T3, C // 2), lambda i: (i, 0)),
        out_shape=jax.ShapeDtypeStruct((N2, C // 2), jnp.int32),
    )(x2, w2t, b2)


NW = 32
QB = 16


@functools.cache
def _interp_call(nq):
    bpw = nq // NW
    nsb = bpw // QB

    def _interp_body(z2_hbm, idxf_hbm, p1f_hbm, p2tf_hbm, out_hbm,
                     idx_v, p1_v, p2t_v, wtmp_v, rows_v, acc_v, sem):
        cc = lax.axis_index("c")
        ss = lax.axis_index("s")
        wid = ss * 2 + cc
        base = wid * bpw
        pltpu.sync_copy(idxf_hbm.at[pl.ds(base * KP, bpw * KP)], idx_v)
        pltpu.sync_copy(p1f_hbm.at[pl.ds(base * 3, bpw * 3)], p1_v)
        pltpu.sync_copy(p2tf_hbm, p2t_v)
        iota16 = lax.broadcasted_iota(jnp.int32, (16,), 0)

        def fire_gathers(b, buf):
            qb = b * QB
            for k in range(K):
                pos = (qb + iota16) * KP + k
                cidx = plsc.load_gather(idx_v, [pos])
                pltpu.async_copy(z2_hbm.at[cidx], rows_v.at[buf, k], sem)

        fire_gathers(0, 0)

        def sb_body(b, carry):
            qb = b * QB
            cur = lax.rem(b, 2)
            for k in range(K):
                pltpu.make_async_copy(
                    z2_hbm.at[pl.ds(0, QB)], rows_v.at[cur, k], sem).wait()
            @pl.when(b + 1 < nsb)
            def _():
                fire_gathers(b + 1, 1 - cur)
            p1x = plsc.load_gather(p1_v, [(qb + iota16) * 3 + 0])
            p1y = plsc.load_gather(p1_v, [(qb + iota16) * 3 + 1])
            p1z = plsc.load_gather(p1_v, [(qb + iota16) * 3 + 2])
            wks = []
            for k in range(K):
                pos = (qb + iota16) * KP + k
                cidx = plsc.load_gather(idx_v, [pos])
                dx = plsc.load_gather(p2t_v, [cidx]) - p1x
                dy = plsc.load_gather(p2t_v, [cidx + N2]) - p1y
                dz = plsc.load_gather(p2t_v, [cidx + 2 * N2]) - p1z
                d16 = (dx * dx + dy * dy) + dz * dz
                d16 = jnp.maximum(d16, jnp.float32(1e-10))
                wks.append(1.0 / d16)
            wsum = ((wks[0] + wks[1]) + (wks[2] + wks[3])) + wks[4]
            for k in range(K):
                wtmp_v[pl.ds(k * 16, 16)] = wks[k] / wsum

            def q_body(q, carry2):
                ws = []
                for k in range(K):
                    wk = plsc.load_gather(
                        wtmp_v, [jnp.broadcast_to(k * 16 + q, (16,))])
                    ws.append(plsc.pack(wk, wk,
                                        format=plsc.PackFormat.INTERLEAVED))
                for j in range(CP // 16):
                    r0 = plsc.bitcast(
                        rows_v[cur, 0, q, pl.ds(j * 16, 16)], jnp.bfloat16)
                    acc = ws[0] * r0
                    for k in range(1, K):
                        rk = plsc.bitcast(
                            rows_v[cur, k, q, pl.ds(j * 16, 16)], jnp.bfloat16)
                        acc = acc + ws[k] * rk
                    acc_v[pl.ds(q * CP + j * 16, 16)] = plsc.bitcast(
                        acc, jnp.int32)
                return carry2

            lax.fori_loop(0, QB, q_body, 0)
            pltpu.sync_copy(acc_v, out_hbm.at[pl.ds((base + qb) * CP, QB * CP)])
            return carry

        lax.fori_loop(0, nsb, sb_body, 0)

    return pl.kernel(
        _interp_body,
        out_type=jax.ShapeDtypeStruct((nq * CP,), jnp.int32),
        mesh=plsc.VectorSubcoreMesh(core_axis_name="c", subcore_axis_name="s"),
        compiler_params=pltpu.CompilerParams(needs_layout_passes=False),
        scratch_types=[
            pltpu.VMEM((bpw * KP,), jnp.int32),
            pltpu.VMEM((bpw * 3,), jnp.float32),
            pltpu.VMEM((3 * N2,), jnp.float32),
            pltpu.VMEM((K * QB,), jnp.float32),
            pltpu.VMEM((2, K, QB, CP), jnp.int32),
            pltpu.VMEM((QB * CP,), jnp.int32),
            pltpu.SemaphoreType.DMA,
        ],
    )


BM_T5 = 512


def _final_body(y_ref, it_ref, st_ref, o_ref):
    mean = st_ref[0:1, :]
    scale = st_ref[1:2, :]
    be = st_ref[2:3, :]
    h1 = jnp.maximum((y_ref[...] - mean) * scale + be, 0.0)
    v = it_ref[...]
    lo = lax.bitcast_convert_type(v << 16, jnp.float32)
    hi = lax.bitcast_convert_type(v & jnp.int32(-65536), jnp.float32)
    o_ref[:, 0:CP] = h1[:, 0:CP] + jnp.maximum(lo, 0.0)
    o_ref[:, CP:C] = h1[:, CP:C] + jnp.maximum(hi, 0.0)


def _final_call(y1, interp, stats):
    return pl.pallas_call(
        _final_body,
        grid=(N1 // BM_T5,),
        in_specs=[
            pl.BlockSpec((BM_T5, C), lambda i: (i, 0)),
            pl.BlockSpec((BM_T5, CP), lambda i: (i, 0)),
            pl.BlockSpec((8, C), lambda i: (0, 0)),
        ],
        out_specs=pl.BlockSpec((BM_T5, C), lambda i: (i, 0)),
        out_shape=jax.ShapeDtypeStruct((N1, C), jnp.float32),
    )(y1, interp, stats)


def kernel(p1, x1, o1, p2, x2, o2, W1, b1, g1, be1, W2, b2):
    p2t = p2.T
    p2tf = p2t.reshape(-1)
    z2 = _lin2_call(x2, W2.T, b2[None, :])
    H = N1 // 2
    halves = []
    for lo in range(0, N1, H):
        p1h = p1[lo:lo + H]
        idx8 = _topk_call(p1h, p2t)
        halves.append(_interp_call(H)(
            z2, idx8.reshape(-1), p1h.reshape(-1), p2tf))
    y1, stats = _lin1_call(x1, W1.T, b1[None, :], g1[None, :], be1[None, :])
    interp = jnp.concatenate(
        [h.reshape(H, CP) for h in halves], axis=0)
    return _final_call(y1, interp, stats)


# speedup vs baseline: 1.1259x; 1.0345x over previous
import functools

import jax
import jax.numpy as jnp
from jax import lax
from jax.experimental import pallas as pl
from jax.experimental.pallas import tpu as pltpu
from jax.experimental.pallas import tpu_sc as plsc

N1 = 16384
N2 = 4096
C = 512
CP = C // 2
K = 5
KP = 8


BN_T1 = 512


def _topk_body(p1_ref, p2t_ref, idx_ref):
    p1 = p1_ref[...]
    p2t = p2t_ref[...]
    mm = jnp.dot(p1.astype(jnp.bfloat16), p2t.astype(jnp.bfloat16),
                 preferred_element_type=jnp.float32)
    s1 = ((p1[:, 0:1] * p1[:, 0:1] + p1[:, 1:2] * p1[:, 1:2])
          + p1[:, 2:3] * p1[:, 2:3])
    s2 = ((p2t[0:1, :] * p2t[0:1, :] + p2t[1:2, :] * p2t[1:2, :])
          + p2t[2:3, :] * p2t[2:3, :])
    d = (s1 - 2.0 * mm) + s2
    iota = lax.broadcasted_iota(jnp.int32, (BN_T1, N2), 1)
    idxs = []
    for k in range(K):
        am = jnp.argmin(d, axis=1).astype(jnp.int32)[:, None]
        idxs.append(am)
        if k < K - 1:
            d = jnp.where(iota == am, jnp.float32(jnp.inf), d)
    ik = jnp.concatenate(idxs, axis=1)
    idx_ref[...] = jnp.concatenate(
        [ik, jnp.zeros((BN_T1, KP - K), jnp.int32)], axis=1)


def _topk_call(p1, p2t):
    nq = p1.shape[0]
    return pl.pallas_call(
        _topk_body,
        grid=(nq // BN_T1,),
        in_specs=[
            pl.BlockSpec((BN_T1, 3), lambda i: (i, 0)),
            pl.BlockSpec((3, N2), lambda i: (0, 0)),
        ],
        out_specs=pl.BlockSpec((BN_T1, KP), lambda i: (i, 0)),
        out_shape=jax.ShapeDtypeStruct((nq, KP), jnp.int32),
    )(p1, p2t)


BM_T2 = 512


def _lin1_body(x1_ref, w1t_ref, b1_ref, g1_ref, be1_ref, y_ref, st_ref):
    i = pl.program_id(0)
    y = jnp.dot(x1_ref[...].astype(jnp.bfloat16),
                w1t_ref[...].astype(jnp.bfloat16),
                preferred_element_type=jnp.float32) + b1_ref[...]
    y_ref[...] = y

    @pl.when(i == 0)
    def _():
        st_ref[...] = jnp.zeros_like(st_ref)

    st_ref[0:1, :] += jnp.sum(y, axis=0, keepdims=True)
    st_ref[1:2, :] += jnp.sum(y * y, axis=0, keepdims=True)

    @pl.when(i == pl.num_programs(0) - 1)
    def _():
        mean = st_ref[0:1, :] / jnp.float32(N1)
        var = st_ref[1:2, :] / jnp.float32(N1) - mean * mean
        st_ref[0:1, :] = mean
        st_ref[1:2, :] = g1_ref[...] * lax.rsqrt(var + jnp.float32(1e-5))
        st_ref[2:3, :] = be1_ref[...]


def _lin1_call(x1, w1t, b1, g1, be1):
    return pl.pallas_call(
        _lin1_body,
        grid=(N1 // BM_T2,),
        in_specs=[
            pl.BlockSpec((BM_T2, 2 * C), lambda i: (i, 0)),
            pl.BlockSpec((2 * C, C), lambda i: (0, 0)),
            pl.BlockSpec((1, C), lambda i: (0, 0)),
            pl.BlockSpec((1, C), lambda i: (0, 0)),
            pl.BlockSpec((1, C), lambda i: (0, 0)),
        ],
        out_specs=[
            pl.BlockSpec((BM_T2, C), lambda i: (i, 0)),
            pl.BlockSpec((8, C), lambda i: (0, 0)),
        ],
        out_shape=[
            jax.ShapeDtypeStruct((N1, C), jnp.float32),
            jax.ShapeDtypeStruct((8, C), jnp.float32),
        ],
    )(x1, w1t, b1, g1, be1)


BM_T3 = 512


def _lin2_body(x2_ref, w2t_ref, b2_ref, z_ref):
    z = jnp.dot(x2_ref[...], w2t_ref[...],
                preferred_element_type=jnp.float32) + b2_ref[...]
    zb = z.astype(jnp.bfloat16)
    lo = lax.bitcast_convert_type(zb[:, :C // 2], jnp.uint16).astype(jnp.int32)
    hi = lax.bitcast_convert_type(zb[:, C // 2:], jnp.uint16).astype(jnp.int32)
    z_ref[...] = lo | (hi << 16)


def _lin2_call(x2, w2t, b2):
    return pl.pallas_call(
        _lin2_body,
        grid=(N2 // BM_T3,),
        in_specs=[
            pl.BlockSpec((BM_T3, C), lambda i: (i, 0)),
            pl.BlockSpec((C, C), lambda i: (0, 0)),
            pl.BlockSpec((1, C), lambda i: (0, 0)),
        ],
        out_specs=pl.BlockSpec((BM_T3, C // 2), lambda i: (i, 0)),
        out_shape=jax.ShapeDtypeStruct((N2, C // 2), jnp.int32),
    )(x2, w2t, b2)


NW = 32
QB = 16


@functools.cache
def _interp_call(nq):
    bpw = nq // NW
    nsb = bpw // QB

    def _interp_body(z2_hbm, idxf_hbm, p1f_hbm, p2tf_hbm, out_hbm,
                     idx_v, p1_v, p2t_v, wtmp_v, rows_v, acc_v, sem):
        cc = lax.axis_index("c")
        ss = lax.axis_index("s")
        wid = ss * 2 + cc
        base = wid * bpw
        pltpu.sync_copy(idxf_hbm.at[pl.ds(base * KP, bpw * KP)], idx_v)
        pltpu.sync_copy(p1f_hbm.at[pl.ds(base * 3, bpw * 3)], p1_v)
        pltpu.sync_copy(p2tf_hbm, p2t_v)
        iota16 = lax.broadcasted_iota(jnp.int32, (16,), 0)

        def fire_gathers(b, buf):
            qb = b * QB
            for k in range(K):
                pos = (qb + iota16) * KP + k
                cidx = plsc.load_gather(idx_v, [pos])
                pltpu.async_copy(z2_hbm.at[cidx], rows_v.at[buf, k], sem)

        fire_gathers(0, 0)

        def sb_body(b, carry):
            qb = b * QB
            cur = lax.rem(b, 2)
            for k in range(K):
                pltpu.make_async_copy(
                    z2_hbm.at[pl.ds(0, QB)], rows_v.at[cur, k], sem).wait()
            @pl.when(b + 1 < nsb)
            def _():
                fire_gathers(b + 1, 1 - cur)
            p1x = plsc.load_gather(p1_v, [(qb + iota16) * 3 + 0])
            p1y = plsc.load_gather(p1_v, [(qb + iota16) * 3 + 1])
            p1z = plsc.load_gather(p1_v, [(qb + iota16) * 3 + 2])
            wks = []
            for k in range(K):
                pos = (qb + iota16) * KP + k
                cidx = plsc.load_gather(idx_v, [pos])
                dx = plsc.load_gather(p2t_v, [cidx]) - p1x
                dy = plsc.load_gather(p2t_v, [cidx + N2]) - p1y
                dz = plsc.load_gather(p2t_v, [cidx + 2 * N2]) - p1z
                d16 = (dx * dx + dy * dy) + dz * dz
                d16 = jnp.maximum(d16, jnp.float32(1e-10))
                wks.append(1.0 / d16)
            wsum = ((wks[0] + wks[1]) + (wks[2] + wks[3])) + wks[4]
            for k in range(K):
                wtmp_v[pl.ds(k * 16, 16)] = wks[k] / wsum

            def q_body(q, carry2):
                ws = []
                for k in range(K):
                    wk = plsc.load_gather(
                        wtmp_v, [jnp.broadcast_to(k * 16 + q, (16,))])
                    ws.append(plsc.pack(wk, wk,
                                        format=plsc.PackFormat.INTERLEAVED))
                for j in range(CP // 16):
                    r0 = plsc.bitcast(
                        rows_v[cur, 0, q, pl.ds(j * 16, 16)], jnp.bfloat16)
                    acc = ws[0] * r0
                    for k in range(1, K):
                        rk = plsc.bitcast(
                            rows_v[cur, k, q, pl.ds(j * 16, 16)], jnp.bfloat16)
                        acc = acc + ws[k] * rk
                    acc_v[pl.ds(q * CP + j * 16, 16)] = plsc.bitcast(
                        acc, jnp.int32)
                return carry2

            lax.fori_loop(0, QB, q_body, 0)
            pltpu.sync_copy(acc_v, out_hbm.at[pl.ds((base + qb) * CP, QB * CP)])
            return carry

        lax.fori_loop(0, nsb, sb_body, 0)

    return pl.kernel(
        _interp_body,
        out_type=jax.ShapeDtypeStruct((nq * CP,), jnp.int32),
        mesh=plsc.VectorSubcoreMesh(core_axis_name="c", subcore_axis_name="s"),
        compiler_params=pltpu.CompilerParams(needs_layout_passes=False),
        scratch_types=[
            pltpu.VMEM((bpw * KP,), jnp.int32),
            pltpu.VMEM((bpw * 3,), jnp.float32),
            pltpu.VMEM((3 * N2,), jnp.float32),
            pltpu.VMEM((K * QB,), jnp.float32),
            pltpu.VMEM((2, K, QB, CP), jnp.int32),
            pltpu.VMEM((QB * CP,), jnp.int32),
            pltpu.SemaphoreType.DMA,
        ],
    )


BM_T5 = 512


def _final_body(y_ref, it_ref, st_ref, o_ref):
    mean = st_ref[0:1, :]
    scale = st_ref[1:2, :]
    be = st_ref[2:3, :]
    h1 = jnp.maximum((y_ref[...] - mean) * scale + be, 0.0)
    v = it_ref[...]
    lo = lax.bitcast_convert_type(v << 16, jnp.float32)
    hi = lax.bitcast_convert_type(v & jnp.int32(-65536), jnp.float32)
    o_ref[:, 0:CP] = h1[:, 0:CP] + jnp.maximum(lo, 0.0)
    o_ref[:, CP:C] = h1[:, CP:C] + jnp.maximum(hi, 0.0)


def _final_call(y1, interp, stats):
    return pl.pallas_call(
        _final_body,
        grid=(N1 // BM_T5,),
        in_specs=[
            pl.BlockSpec((BM_T5, C), lambda i: (i, 0)),
            pl.BlockSpec((BM_T5, CP), lambda i: (i, 0)),
            pl.BlockSpec((8, C), lambda i: (0, 0)),
        ],
        out_specs=pl.BlockSpec((BM_T5, C), lambda i: (i, 0)),
        out_shape=jax.ShapeDtypeStruct((N1, C), jnp.float32),
    )(y1, interp, stats)


def kernel(p1, x1, o1, p2, x2, o2, W1, b1, g1, be1, W2, b2):
    p2t = p2.T
    p2tf = p2t.reshape(-1)
    z2 = _lin2_call(x2, W2.T, b2[None, :])
    H = N1 // 2
    halves = []
    for lo in range(0, N1, H):
        p1h = p1[lo:lo + H]
        idx8 = _topk_call(p1h, p2t)
        halves.append(_interp_call(H)(
            z2, idx8.reshape(-1), p1h.reshape(-1), p2tf))
    y1, stats = _lin1_call(x1, W1.T, b1[None, :], g1[None, :], be1[None, :])
    interp = jnp.concatenate(
        [h.reshape(H, CP) for h in halves], axis=0)
    return _final_call(y1, interp, stats)


# speedup vs baseline: 1.2471x; 1.1076x over previous
import functools

import jax
import jax.numpy as jnp
from jax import lax
from jax.experimental import pallas as pl
from jax.experimental.pallas import tpu as pltpu
from jax.experimental.pallas import tpu_sc as plsc

N1 = 16384
N2 = 4096
C = 512
CP = C // 2
K = 5
KP = 8


BN_T1 = 512


def _topk_body(p1_ref, p2t_ref, idx_ref):
    p1 = p1_ref[...]
    p2t = p2t_ref[...]
    mm = jnp.dot(p1.astype(jnp.bfloat16), p2t.astype(jnp.bfloat16),
                 preferred_element_type=jnp.float32)
    s1 = ((p1[:, 0:1] * p1[:, 0:1] + p1[:, 1:2] * p1[:, 1:2])
          + p1[:, 2:3] * p1[:, 2:3])
    s2 = ((p2t[0:1, :] * p2t[0:1, :] + p2t[1:2, :] * p2t[1:2, :])
          + p2t[2:3, :] * p2t[2:3, :])
    d = (s1 - 2.0 * mm) + s2
    iota = lax.broadcasted_iota(jnp.int32, (BN_T1, N2), 1)
    idxs = []
    for k in range(K):
        am = jnp.argmin(d, axis=1).astype(jnp.int32)[:, None]
        idxs.append(am)
        if k < K - 1:
            d = jnp.where(iota == am, jnp.float32(jnp.inf), d)
    ik = jnp.concatenate(idxs, axis=1)
    idx_ref[...] = jnp.concatenate(
        [ik, jnp.zeros((BN_T1, KP - K), jnp.int32)], axis=1)


def _topk_call(p1, p2t):
    nq = p1.shape[0]
    return pl.pallas_call(
        _topk_body,
        grid=(nq // BN_T1,),
        in_specs=[
            pl.BlockSpec((BN_T1, 3), lambda i: (i, 0)),
            pl.BlockSpec((3, N2), lambda i: (0, 0)),
        ],
        out_specs=pl.BlockSpec((BN_T1, KP), lambda i: (i, 0)),
        out_shape=jax.ShapeDtypeStruct((nq, KP), jnp.int32),
    )(p1, p2t)


BM_T2 = 512


def _lin1_body(x1_ref, w1t_ref, b1_ref, g1_ref, be1_ref, y_ref, st_ref):
    i = pl.program_id(0)
    y = jnp.dot(x1_ref[...].astype(jnp.bfloat16),
                w1t_ref[...].astype(jnp.bfloat16),
                preferred_element_type=jnp.float32) + b1_ref[...]
    y_ref[...] = y

    @pl.when(i == 0)
    def _():
        st_ref[...] = jnp.zeros_like(st_ref)

    st_ref[0:1, :] += jnp.sum(y, axis=0, keepdims=True)
    st_ref[1:2, :] += jnp.sum(y * y, axis=0, keepdims=True)

    @pl.when(i == pl.num_programs(0) - 1)
    def _():
        mean = st_ref[0:1, :] / jnp.float32(N1)
        var = st_ref[1:2, :] / jnp.float32(N1) - mean * mean
        st_ref[0:1, :] = mean
        st_ref[1:2, :] = g1_ref[...] * lax.rsqrt(var + jnp.float32(1e-5))
        st_ref[2:3, :] = be1_ref[...]


def _lin1_call(x1, w1t, b1, g1, be1):
    return pl.pallas_call(
        _lin1_body,
        grid=(N1 // BM_T2,),
        in_specs=[
            pl.BlockSpec((BM_T2, 2 * C), lambda i: (i, 0)),
            pl.BlockSpec((2 * C, C), lambda i: (0, 0)),
            pl.BlockSpec((1, C), lambda i: (0, 0)),
            pl.BlockSpec((1, C), lambda i: (0, 0)),
            pl.BlockSpec((1, C), lambda i: (0, 0)),
        ],
        out_specs=[
            pl.BlockSpec((BM_T2, C), lambda i: (i, 0)),
            pl.BlockSpec((8, C), lambda i: (0, 0)),
        ],
        out_shape=[
            jax.ShapeDtypeStruct((N1, C), jnp.float32),
            jax.ShapeDtypeStruct((8, C), jnp.float32),
        ],
    )(x1, w1t, b1, g1, be1)


BM_T3 = 512


def _lin2_body(x2_ref, w2t_ref, b2_ref, z_ref):
    z = jnp.dot(x2_ref[...], w2t_ref[...],
                preferred_element_type=jnp.float32) + b2_ref[...]
    zb = z.astype(jnp.bfloat16)
    lo = lax.bitcast_convert_type(zb[:, :C // 2], jnp.uint16).astype(jnp.int32)
    hi = lax.bitcast_convert_type(zb[:, C // 2:], jnp.uint16).astype(jnp.int32)
    z_ref[...] = lo | (hi << 16)


def _lin2_call(x2, w2t, b2):
    return pl.pallas_call(
        _lin2_body,
        grid=(N2 // BM_T3,),
        in_specs=[
            pl.BlockSpec((BM_T3, C), lambda i: (i, 0)),
            pl.BlockSpec((C, C), lambda i: (0, 0)),
            pl.BlockSpec((1, C), lambda i: (0, 0)),
        ],
        out_specs=pl.BlockSpec((BM_T3, C // 2), lambda i: (i, 0)),
        out_shape=jax.ShapeDtypeStruct((N2, C // 2), jnp.int32),
    )(x2, w2t, b2)


NW = 32
QB = 16


@functools.cache
def _interp_call(nq):
    bpw = nq // NW
    nsb = bpw // QB

    def _interp_body(z2_hbm, idx_hbm, p1_hbm, p2t_hbm, out_hbm,
                     idx_v, p1_v, p2t_v, wtmp_v, rows_v, acc_v, sem):
        cc = lax.axis_index("c")
        ss = lax.axis_index("s")
        wid = ss * 2 + cc
        base = wid * bpw
        pltpu.sync_copy(idx_hbm.at[pl.ds(base, bpw), :], idx_v)
        pltpu.sync_copy(p1_hbm.at[pl.ds(base, bpw), :], p1_v)
        pltpu.sync_copy(p2t_hbm, p2t_v)
        iota16 = lax.broadcasted_iota(jnp.int32, (16,), 0)

        def col(k):
            return jnp.full((16,), k, jnp.int32)

        def fire_gathers(b, buf):
            qb = b * QB
            for k in range(K):
                cidx = plsc.load_gather(idx_v, [qb + iota16, col(k)])
                pltpu.async_copy(z2_hbm.at[cidx], rows_v.at[buf, k], sem)

        fire_gathers(0, 0)

        def sb_body(b, carry):
            qb = b * QB
            cur = lax.rem(b, 2)
            for k in range(K):
                pltpu.make_async_copy(
                    z2_hbm.at[pl.ds(0, QB)], rows_v.at[cur, k], sem).wait()
            @pl.when(b + 1 < nsb)
            def _():
                fire_gathers(b + 1, 1 - cur)
            p1x = plsc.load_gather(p1_v, [qb + iota16, col(0)])
            p1y = plsc.load_gather(p1_v, [qb + iota16, col(1)])
            p1z = plsc.load_gather(p1_v, [qb + iota16, col(2)])
            wks = []
            for k in range(K):
                cidx = plsc.load_gather(idx_v, [qb + iota16, col(k)])
                dx = plsc.load_gather(p2t_v, [col(0), cidx]) - p1x
                dy = plsc.load_gather(p2t_v, [col(1), cidx]) - p1y
                dz = plsc.load_gather(p2t_v, [col(2), cidx]) - p1z
                d16 = (dx * dx + dy * dy) + dz * dz
                d16 = jnp.maximum(d16, jnp.float32(1e-10))
                wks.append(1.0 / d16)
            wsum = ((wks[0] + wks[1]) + (wks[2] + wks[3])) + wks[4]
            for k in range(K):
                wtmp_v[pl.ds(k * 16, 16)] = wks[k] / wsum

            def q_body(q, carry2):
                ws = []
                for k in range(K):
                    wk = plsc.load_gather(
                        wtmp_v, [jnp.broadcast_to(k * 16 + q, (16,))])
                    ws.append(plsc.pack(wk, wk,
                                        format=plsc.PackFormat.INTERLEAVED))
                for j in range(CP // 16):
                    r0 = plsc.bitcast(
                        rows_v[cur, 0, q, pl.ds(j * 16, 16)], jnp.bfloat16)
                    acc = ws[0] * r0
                    for k in range(1, K):
                        rk = plsc.bitcast(
                            rows_v[cur, k, q, pl.ds(j * 16, 16)], jnp.bfloat16)
                        acc = acc + ws[k] * rk
                    acc_v[q, pl.ds(j * 16, 16)] = plsc.bitcast(
                        acc, jnp.int32)
                return carry2

            lax.fori_loop(0, QB, q_body, 0)
            pltpu.sync_copy(acc_v, out_hbm.at[pl.ds(base + qb, QB), :])
            return carry

        lax.fori_loop(0, nsb, sb_body, 0)

    return pl.kernel(
        _interp_body,
        out_type=jax.ShapeDtypeStruct((nq, CP), jnp.int32),
        mesh=plsc.VectorSubcoreMesh(core_axis_name="c", subcore_axis_name="s"),
        compiler_params=pltpu.CompilerParams(needs_layout_passes=False),
        scratch_types=[
            pltpu.VMEM((bpw, KP), jnp.int32),
            pltpu.VMEM((bpw, 3), jnp.float32),
            pltpu.VMEM((3, N2), jnp.float32),
            pltpu.VMEM((K * QB,), jnp.float32),
            pltpu.VMEM((2, K, QB, CP), jnp.int32),
            pltpu.VMEM((QB, CP), jnp.int32),
            pltpu.SemaphoreType.DMA,
        ],
    )


BM_T5 = 512


_HBLK = (N1 // 2) // BM_T5


def _final_body(y_ref, ita_ref, itb_ref, st_ref, o_ref):
    i = pl.program_id(0)
    mean = st_ref[0:1, :]
    scale = st_ref[1:2, :]
    be = st_ref[2:3, :]
    h1 = jnp.maximum((y_ref[...] - mean) * scale + be, 0.0)
    v = jnp.where(i < _HBLK, ita_ref[...], itb_ref[...])
    lo = lax.bitcast_convert_type(v << 16, jnp.float32)
    hi = lax.bitcast_convert_type(v & jnp.int32(-65536), jnp.float32)
    o_ref[:, 0:CP] = h1[:, 0:CP] + jnp.maximum(lo, 0.0)
    o_ref[:, CP:C] = h1[:, CP:C] + jnp.maximum(hi, 0.0)


def _final_call(y1, interp_a, interp_b, stats):
    return pl.pallas_call(
        _final_body,
        grid=(N1 // BM_T5,),
        in_specs=[
            pl.BlockSpec((BM_T5, C), lambda i: (i, 0)),
            pl.BlockSpec((BM_T5, CP), lambda i: (jnp.minimum(i, _HBLK - 1), 0)),
            pl.BlockSpec((BM_T5, CP), lambda i: (jnp.maximum(i - _HBLK, 0), 0)),
            pl.BlockSpec((8, C), lambda i: (0, 0)),
        ],
        out_specs=pl.BlockSpec((BM_T5, C), lambda i: (i, 0)),
        out_shape=jax.ShapeDtypeStruct((N1, C), jnp.float32),
    )(y1, interp_a, interp_b, stats)


def kernel(p1, x1, o1, p2, x2, o2, W1, b1, g1, be1, W2, b2):
    p2t = p2.T
    z2 = _lin2_call(x2, W2.T, b2[None, :])
    H = N1 // 2
    halves = []
    for lo in range(0, N1, H):
        p1h = p1[lo:lo + H]
        idx8 = _topk_call(p1h, p2t)
        halves.append(_interp_call(H)(z2, idx8, p1h, p2t))
    y1, stats = _lin1_call(x1, W1.T, b1[None, :], g1[None, :], be1[None, :])
    return _final_call(y1, halves[0], halves[1], stats)
